# baseline (speedup 1.0000x reference)
import jax
import jax.numpy as jnp
from jax.experimental import pallas as pl
from jax.experimental.pallas import tpu as pltpu

_N = 8192
_D = 32
_R = 64
_B = 2048
_NB = _N // _B
_EPS = 1e-5
_CLAMP = 64e-6
_LOG2E = 1.4426950408889634


def _body(zt_ref, wq, wk, wv, wo, w1t, w2, om, bq, bk, bv, bo, b1, b2,
          g1, be1, g2, be2, out_ref, phi_ref, sf_ref):
    f32 = jnp.float32
    dg = lambda a, b, dims: jax.lax.dot_general(
        a, b, (dims, ((), ())), preferred_element_type=f32)
    s = pl.program_id(0)
    j = pl.program_id(1)

    @pl.when(jnp.logical_and(s == 0, j == 0))
    def _init():
        sf_ref[...] = jnp.zeros_like(sf_ref)

    @pl.when(s == 0)
    def _features():
        ZT = zt_ref[...]
        W4 = jnp.concatenate([wq[...], wk[...], wv[...]], axis=0)
        b4 = jnp.concatenate([bq[...], bk[...], bv[...]], axis=1).T
        QKVV = dg(W4, ZT, ((1,), (0,))) + b4

        col = jax.lax.broadcasted_iota(jnp.int32, (2, 3 * _D), 1)
        rowi = jax.lax.broadcasted_iota(jnp.int32, (2, 3 * _D), 0)
        O2 = jnp.where((col >= rowi * _D) & (col < (rowi + 1) * _D),
                       jnp.float32(1.0), jnp.float32(0.0))
        N2 = dg(O2, QKVV * QKVV, ((1,), (0,)))
        T2 = jnp.minimum(jax.lax.rsqrt(N2), 1e6) * _LOG2E
        T2b = jnp.concatenate([
            jnp.broadcast_to(T2[0:1, :], (_D, _B)),
            jnp.broadcast_to(T2[1:2, :], (_D, _B))], axis=0)
        QKn = QKVV[0:2 * _D, :] * T2b

        OMT = om[...].T
        zo = jnp.zeros((_R, _D), f32)
        GT = jnp.concatenate([
            jnp.concatenate([OMT, zo], axis=1),
            jnp.concatenate([zo, OMT], axis=1)], axis=0)
        PHI = jnp.exp2(dg(GT, QKn, ((1,), (0,))))
        phi_ref[:, pl.ds(j * _B, _B)] = PHI
        sf_ref[...] += dg(PHI, QKVV, ((1,), (1,)))

    @pl.when(s == 1)
    def _emit():
        Sf = sf_ref[...]
        S = Sf[_R:128, 2 * _D:3 * _D]
        SWoT = dg(wo[...], S, ((1,), (1,)))
        SsT = dg(jnp.ones((1, _D), f32), S, ((1,), (1,)))
        M2T = jnp.concatenate([
            jnp.concatenate([SWoT, SsT], axis=0),
            jnp.zeros((_D + 1, _R), f32)], axis=1)

        PHI = phi_ref[:, pl.ds(j * _B, _B)]
        U = dg(M2T, PHI, ((1,), (0,)))
        t = 1.0 / jnp.maximum(U[_D:_D + 1, :], _CLAMP)
        X = zt_ref[...] + U[0:_D, :] * t + bo[...].T

        O32 = jnp.full((1, _D), 1.0 / _D, f32)
        mu = dg(O32, X, ((1,), (0,)))
        m2 = dg(O32, X * X, ((1,), (0,)))
        a1 = jax.lax.rsqrt(m2 - mu * mu + _EPS)
        XN = X * a1 - mu * a1
        Z1 = XN * g1[...].T + be1[...].T

        hid = jnp.maximum(dg(w1t[...], Z1, ((0,), (0,))) + b1[...].T, 0.0)
        Y = Z1 + dg(w2[...], hid, ((1,), (0,))) + b2[...].T

        mu2 = dg(O32, Y, ((1,), (0,)))
        m22 = dg(O32, Y * Y, ((1,), (0,)))
        a2 = jax.lax.rsqrt(m22 - mu2 * mu2 + _EPS)
        out_ref[...] = (Y * a2 - mu2 * a2) * g2[...].T + be2[...].T


@jax.jit
def kernel(Z, Wq, bq, Wk, bk, Wv, bv, Wo, bo, W1, b1, W2, b2,
           g1, beta1, g2, beta2, omega):
    rowv = lambda v: v.reshape(1, -1)
    args = (Z.T, Wq, Wk, Wv, Wo, W1.T, W2, omega,
            rowv(bq), rowv(bk), rowv(bv), rowv(bo), rowv(b1), rowv(b2),
            rowv(g1), rowv(beta1), rowv(g2), rowv(beta2))
    blk = pl.BlockSpec((_D, _B), lambda s, j: (0, j))
    in_specs = [blk]
    for a in args[1:]:
        in_specs.append(pl.BlockSpec(a.shape, lambda s, j: (0, 0)))
    out_t = pl.pallas_call(
        _body,
        grid=(2, _NB),
        in_specs=in_specs,
        out_specs=blk,
        out_shape=jax.ShapeDtypeStruct((_D, _N), jnp.float32),
        scratch_shapes=[pltpu.VMEM((128, _N), jnp.float32),
                        pltpu.VMEM((128, 3 * _D), jnp.float32)],
        compiler_params=pltpu.CompilerParams(
            dimension_semantics=("arbitrary", "arbitrary")),
    )(*args)
    return out_t.T


# speedup vs baseline: 1.4100x; 1.4100x over previous
import jax
import jax.numpy as jnp
from jax.experimental import pallas as pl
from jax.experimental.pallas import tpu as pltpu

_N = 8192
_D = 32
_R = 64
_B = 4096
_NB = _N // _B
_EPS = 1e-5
_CLAMP = 64e-6
_LOG2E = 1.4426950408889634


def _body(zt_ref, wq, wk, wv, wo, w1t, w2, om, bq, bk, bv, bo, b1, b2,
          g1, be1, g2, be2, out_ref, phi_ref, sf_ref):
    f32 = jnp.float32
    dg = lambda a, b, dims: jax.lax.dot_general(
        a, b, (dims, ((), ())), preferred_element_type=f32)
    s = pl.program_id(0)
    j = pl.program_id(1)

    @pl.when(jnp.logical_and(s == 0, j == 0))
    def _init():
        sf_ref[...] = jnp.zeros_like(sf_ref)

    @pl.when(s == 0)
    def _features():
        ZT = zt_ref[...]
        W4 = jnp.concatenate([wq[...], wk[...], wv[...]], axis=0)
        b4 = jnp.concatenate([bq[...], bk[...], bv[...]], axis=1).T
        QKVV = dg(W4, ZT, ((1,), (0,))) + b4

        col = jax.lax.broadcasted_iota(jnp.int32, (2, 3 * _D), 1)
        rowi = jax.lax.broadcasted_iota(jnp.int32, (2, 3 * _D), 0)
        O2 = jnp.where((col >= rowi * _D) & (col < (rowi + 1) * _D),
                       jnp.float32(1.0), jnp.float32(0.0))
        N2 = dg(O2, QKVV * QKVV, ((1,), (0,)))
        T2 = jnp.minimum(jax.lax.rsqrt(N2), 1e6) * _LOG2E
        T2b = jnp.concatenate([
            jnp.broadcast_to(T2[0:1, :], (_D, _B)),
            jnp.broadcast_to(T2[1:2, :], (_D, _B))], axis=0)
        QKn = QKVV[0:2 * _D, :] * T2b

        OMT = om[...].T
        zo = jnp.zeros((_R, _D), f32)
        GT = jnp.concatenate([
            jnp.concatenate([OMT, zo], axis=1),
            jnp.concatenate([zo, OMT], axis=1)], axis=0)
        PHI = jnp.exp2(dg(GT, QKn, ((1,), (0,))))
        phi_ref[:, pl.ds(j * _B, _B)] = PHI
        sf_ref[...] += dg(PHI, QKVV, ((1,), (1,)))

    @pl.when(s == 1)
    def _emit():
        Sf = sf_ref[...]
        S = Sf[_R:128, 2 * _D:3 * _D]
        SWoT = dg(wo[...], S, ((1,), (1,)))
        SsT = dg(jnp.ones((1, _D), f32), S, ((1,), (1,)))
        M2T = jnp.concatenate([
            jnp.concatenate([SWoT, SsT], axis=0),
            jnp.zeros((_D + 1, _R), f32)], axis=1)

        PHI = phi_ref[:, pl.ds(j * _B, _B)]
        U = dg(M2T, PHI, ((1,), (0,)))
        t = 1.0 / jnp.maximum(U[_D:_D + 1, :], _CLAMP)
        X = zt_ref[...] + U[0:_D, :] * t + bo[...].T

        O32 = jnp.full((1, _D), 1.0 / _D, f32)
        mu = dg(O32, X, ((1,), (0,)))
        m2 = dg(O32, X * X, ((1,), (0,)))
        a1 = jax.lax.rsqrt(m2 - mu * mu + _EPS)
        XN = X * a1 - mu * a1
        Z1 = XN * g1[...].T + be1[...].T

        hid = jnp.maximum(dg(w1t[...], Z1, ((0,), (0,))) + b1[...].T, 0.0)
        Y = Z1 + dg(w2[...], hid, ((1,), (0,))) + b2[...].T

        mu2 = dg(O32, Y, ((1,), (0,)))
        m22 = dg(O32, Y * Y, ((1,), (0,)))
        a2 = jax.lax.rsqrt(m22 - mu2 * mu2 + _EPS)
        out_ref[...] = (Y * a2 - mu2 * a2) * g2[...].T + be2[...].T


@jax.jit
def kernel(Z, Wq, bq, Wk, bk, Wv, bv, Wo, bo, W1, b1, W2, b2,
           g1, beta1, g2, beta2, omega):
    rowv = lambda v: v.reshape(1, -1)
    args = (Z.T, Wq, Wk, Wv, Wo, W1.T, W2, omega,
            rowv(bq), rowv(bk), rowv(bv), rowv(bo), rowv(b1), rowv(b2),
            rowv(g1), rowv(beta1), rowv(g2), rowv(beta2))
    blk = pl.BlockSpec((_D, _B), lambda s, j: (0, j))
    in_specs = [blk]
    for a in args[1:]:
        in_specs.append(pl.BlockSpec(a.shape, lambda s, j: (0, 0)))
    out_t = pl.pallas_call(
        _body,
        grid=(2, _NB),
        in_specs=in_specs,
        out_specs=blk,
        out_shape=jax.ShapeDtypeStruct((_D, _N), jnp.float32),
        scratch_shapes=[pltpu.VMEM((128, _N), jnp.float32),
                        pltpu.VMEM((128, 3 * _D), jnp.float32)],
        compiler_params=pltpu.CompilerParams(
            dimension_semantics=("arbitrary", "arbitrary")),
    )(*args)
    return out_t.T


# speedup vs baseline: 1.4619x; 1.0368x over previous
import jax
import jax.numpy as jnp
from jax.experimental import pallas as pl

_N = 8192
_D = 32
_R = 64
_EPS = 1e-5
_CLAMP = 64e-6


def _body(zt_ref, wq, wk, wv, wo, w1t, w2, om, bq, bk, bv, bo, b1, b2,
          g1, be1, g2, be2, out_ref):
    f32 = jnp.float32
    dg = lambda a, b, dims: jax.lax.dot_general(
        a, b, (dims, ((), ())), preferred_element_type=f32)

    ZT = zt_ref[...]

    W4 = jnp.concatenate([wq[...], wk[...], wv[...]], axis=0)
    b4 = jnp.concatenate([bq[...], bk[...], bv[...]], axis=1).T
    QKVV = dg(W4, ZT, ((1,), (0,))) + b4

    col = jax.lax.broadcasted_iota(jnp.int32, (2, 3 * _D), 1)
    rowi = jax.lax.broadcasted_iota(jnp.int32, (2, 3 * _D), 0)
    O2 = jnp.where((col >= rowi * _D) & (col < (rowi + 1) * _D),
                   jnp.float32(1.0), jnp.float32(0.0))
    N2 = dg(O2, QKVV * QKVV, ((1,), (0,)))
    T2 = jnp.minimum(jax.lax.rsqrt(N2), 1e6) * 1.4426950408889634
    T2b = jnp.concatenate([
        jnp.broadcast_to(T2[0:1, :], (_D, _N)),
        jnp.broadcast_to(T2[1:2, :], (_D, _N))], axis=0)
    QKn = QKVV[0:2 * _D, :] * T2b

    OMT = om[...].T
    zo = jnp.zeros((_R, _D), f32)
    GT = jnp.concatenate([
        jnp.concatenate([OMT, zo], axis=1),
        jnp.concatenate([zo, OMT], axis=1)], axis=0)
    P = dg(GT.astype(jnp.bfloat16), QKn.astype(jnp.bfloat16), ((1,), (0,)))
    PHI = jnp.exp2(P)

    Sf = dg(PHI.astype(jnp.bfloat16), QKVV.astype(jnp.bfloat16), ((1,), (1,)))
    S = Sf[_R:128, 2 * _D:3 * _D]
    SWoT = dg(wo[...], S, ((1,), (1,)))
    ones32 = jnp.ones((1, _D), f32)
    SsT = dg(ones32, S, ((1,), (1,)))
    M2T = jnp.concatenate([
        jnp.concatenate([SWoT, SsT], axis=0),
        jnp.zeros((_D + 1, _R), f32)], axis=1)

    U = dg(M2T, PHI, ((1,), (0,)))
    t = 1.0 / jnp.maximum(U[_D:_D + 1, :], _CLAMP)
    X = ZT + U[0:_D, :] * t + bo[...].T

    O32 = jnp.full((1, _D), 1.0 / _D, f32)
    mu = dg(O32, X, ((1,), (0,)))
    m2 = dg(O32, X * X, ((1,), (0,)))
    a1 = jax.lax.rsqrt(m2 - mu * mu + _EPS)
    XN = X * a1 - mu * a1
    Z1 = XN * g1[...].T + be1[...].T

    hid = jnp.maximum(dg(w1t[...].astype(jnp.bfloat16), Z1.astype(jnp.bfloat16), ((0,), (0,))) + b1[...].T, 0.0)
    Y = Z1 + dg(w2[...], hid, ((1,), (0,))) + b2[...].T

    mu2 = dg(O32, Y, ((1,), (0,)))
    m22 = dg(O32, Y * Y, ((1,), (0,)))
    a2 = jax.lax.rsqrt(m22 - mu2 * mu2 + _EPS)
    out_ref[...] = (Y * a2 - mu2 * a2) * g2[...].T + be2[...].T


@jax.jit
def kernel(Z, Wq, bq, Wk, bk, Wv, bv, Wo, bo, W1, b1, W2, b2,
           g1, beta1, g2, beta2, omega):
    rowv = lambda v: v.reshape(1, -1)
    args = (Z.T, Wq, Wk, Wv, Wo, W1.T, W2, omega,
            rowv(bq), rowv(bk), rowv(bv), rowv(bo), rowv(b1), rowv(b2),
            rowv(g1), rowv(beta1), rowv(g2), rowv(beta2))
    out_t = pl.pallas_call(
        _body,
        out_shape=jax.ShapeDtypeStruct((_D, _N), jnp.float32),
    )(*args)
    return out_t.T


# speedup vs baseline: 1.4801x; 1.0125x over previous
import jax
import jax.numpy as jnp
from jax.experimental import pallas as pl

_N = 8192
_D = 32
_R = 64
_EPS = 1e-5
_CLAMP = 64e-6


def _body(zt_ref, wq, wk, wv, wo, w1t, w2, om, bq, bk, bv, bo, b1, b2,
          g1, be1, g2, be2, out_ref):
    f32 = jnp.float32
    dg = lambda a, b, dims: jax.lax.dot_general(
        a, b, (dims, ((), ())), preferred_element_type=f32)

    ZT = zt_ref[...]

    W4 = jnp.concatenate([wq[...], wk[...], wv[...]], axis=0)
    b4 = jnp.concatenate([bq[...], bk[...], bv[...]], axis=1).T
    QKVV = dg(W4, ZT, ((1,), (0,))) + b4

    col = jax.lax.broadcasted_iota(jnp.int32, (2, 3 * _D), 1)
    rowi = jax.lax.broadcasted_iota(jnp.int32, (2, 3 * _D), 0)
    O2 = jnp.where((col >= rowi * _D) & (col < (rowi + 1) * _D),
                   jnp.float32(1.0), jnp.float32(0.0))
    N2 = dg(O2, QKVV * QKVV, ((1,), (0,)))
    T2 = jnp.minimum(jax.lax.rsqrt(N2), 1e6) * 1.4426950408889634
    T2b = jnp.concatenate([
        jnp.broadcast_to(T2[0:1, :], (_D, _N)),
        jnp.broadcast_to(T2[1:2, :], (_D, _N))], axis=0)
    QKn = QKVV[0:2 * _D, :] * T2b

    OMT = om[...].T
    zo = jnp.zeros((_R, _D), f32)
    GT = jnp.concatenate([
        jnp.concatenate([OMT, zo], axis=1),
        jnp.concatenate([zo, OMT], axis=1)], axis=0)
    P = dg(GT, QKn, ((1,), (0,)))
    PHI = jnp.exp2(P)

    Sf = dg(PHI, QKVV, ((1,), (1,)))
    S = Sf[_R:128, 2 * _D:3 * _D]
    SWoT = dg(wo[...], S, ((1,), (1,)))
    ones32 = jnp.ones((1, _D), f32)
    SsT = dg(ones32, S, ((1,), (1,)))
    M2T = jnp.concatenate([
        jnp.concatenate([SWoT, SsT], axis=0),
        jnp.zeros((_D + 1, _R), f32)], axis=1)

    U = dg(M2T, PHI, ((1,), (0,)))
    t = 1.0 / jnp.maximum(U[_D:_D + 1, :], _CLAMP)
    X = ZT + U[0:_D, :] * t + bo[...].T

    O32 = jnp.full((1, _D), 1.0 / _D, f32)
    mu = dg(O32, X, ((1,), (0,)))
    m2 = dg(O32, X * X, ((1,), (0,)))
    a1 = jax.lax.rsqrt(m2 - mu * mu + _EPS)
    XN = X * a1 - mu * a1
    Z1 = XN * g1[...].T + be1[...].T

    hid = jnp.maximum(dg(w1t[...], Z1, ((0,), (0,))) + b1[...].T, 0.0)
    Y = Z1 + dg(w2[...], hid, ((1,), (0,))) + b2[...].T

    mu2 = dg(O32, Y, ((1,), (0,)))
    m22 = dg(O32, Y * Y, ((1,), (0,)))
    a2 = jax.lax.rsqrt(m22 - mu2 * mu2 + _EPS)
    out_ref[...] = (Y * a2 - mu2 * a2) * g2[...].T + be2[...].T


@jax.jit
def kernel(Z, Wq, bq, Wk, bk, Wv, bv, Wo, bo, W1, b1, W2, b2,
           g1, beta1, g2, beta2, omega):
    rowv = lambda v: v.reshape(1, -1)
    args = (Z.T, Wq, Wk, Wv, Wo, W1.T, W2, omega,
            rowv(bq), rowv(bk), rowv(bv), rowv(bo), rowv(b1), rowv(b2),
            rowv(g1), rowv(beta1), rowv(g2), rowv(beta2))
    out_t = pl.pallas_call(
        _body,
        out_shape=jax.ShapeDtypeStruct((_D, _N), jnp.float32),
    )(*args)
    return out_t.T
